# Initial kernel scaffold; baseline (speedup 1.0000x reference)
import jax
import jax.numpy as jnp
from jax.experimental import pallas as pl


def kernel(x, W1, g1, b1, W2, g2, b2, W3, g3, b3, W4, g4, b4, W5, g5, b5, fc1_w, fc1_b, bn1_g, bn1_b, fc2_w, fc2_b, bn2_g, bn2_b, fc3_w, fc3_b):
    raise NotImplementedError("write your pallas kernel here")


# speedup vs baseline: 6.3068x; 6.3068x over previous
import jax
import jax.numpy as jnp
from jax import lax
from jax.experimental import pallas as pl

_B, _N, _K, _NUM_CLASS = 8, 1024, 20, 40
_EPS = 1e-5
_F32 = jnp.float32


def _rup(c, m=8):
    return (c + m - 1) // m * m


def _lrelu(v):
    return jnp.where(v >= 0, v, 0.2 * v)


_DNT = (((1,), (1,)), ((), ()))
_DNS = (((1,), (0,)), ((), ()))
_BF16 = jnp.bfloat16


def _d_tt(p, q):
    return lax.dot_general(p, q, _DNT, preferred_element_type=_F32)


def _d_std(p, q):
    return lax.dot_general(p, q, _DNS, preferred_element_type=_F32)


def _split4(v):
    def top(x):
        return lax.bitcast_convert_type(
            lax.bitcast_convert_type(x, jnp.int32) & (-65536), _F32)

    h32 = top(v)
    r1 = v - h32
    m32 = top(r1)
    r2 = r1 - m32
    l32 = top(r2)
    q32 = r2 - l32
    return (h32.astype(_BF16), m32.astype(_BF16), l32.astype(_BF16),
            q32.astype(_BF16))


def _edge_layer_call(xp, xpt, xxc, xxr, wf, C):
    Cp = xp.shape[1]
    o, P2 = wf.shape

    def kern(x_ref, xt_ref, xxc_ref, xxr_ref, wf_ref, zmax_ref,
             zmin_ref, t1_ref, t2_ref):
        xb = x_ref[0]
        xt = xt_ref[0]
        xtb = xt.astype(_BF16)
        G = _d_tt(xtb, xtb)
        t = xxc_ref[0][:, 0:1] + xxr_ref[0][0:1, :]
        dist = jnp.maximum(t - 2.0 * G, 0.0)
        neg = -dist
        cols = lax.broadcasted_iota(jnp.int32, (_N, _N), 1)

        wf_b = wf_ref[...].astype(_BF16)
        xbb = xb[:C, :].astype(_BF16)
        xh, xm, xl, xq = _split4(xb)

        S = jnp.zeros((o, _N), _F32)
        Q = jnp.zeros((o, _N), _F32)
        Mx = jnp.full((o, _N), -jnp.inf, _F32)
        Mn = jnp.full((o, _N), jnp.inf, _F32)
        for k in range(_K):
            m = jnp.max(neg, axis=1, keepdims=True)
            idxk = jnp.min(jnp.where(neg == m, cols, _N), axis=1,
                           keepdims=True)
            P = (cols == idxk)
            if k < _K - 1:
                neg = jnp.where(P, -jnp.inf, neg)
            pb = P.astype(_BF16)
            nk = ((_d_tt(xh, pb) + _d_tt(xm, pb))
                  + (_d_tt(xl, pb) + _d_tt(xq, pb)))
            dk = (nk - xb)[:C, :].astype(_BF16)
            if P2 > 2 * C:
                ek = jnp.concatenate(
                    [dk, xbb, jnp.zeros((P2 - 2 * C, _N), _BF16)], axis=0)
            else:
                ek = jnp.concatenate([dk, xbb], axis=0)
            ak = _d_std(wf_b, ek)
            S = S + ak
            Q = Q + ak * ak
            Mx = jnp.maximum(Mx, ak)
            Mn = jnp.minimum(Mn, ak)

        zmax_ref[0] = Mx
        zmin_ref[0] = Mn
        t1 = jnp.sum(S, axis=1, keepdims=True)
        t2 = jnp.sum(Q, axis=1, keepdims=True)
        t1_ref[0] = jnp.broadcast_to(t1, (o, 128))
        t2_ref[0] = jnp.broadcast_to(t2, (o, 128))

    return pl.pallas_call(
        kern,
        grid=(_B,),
        in_specs=[
            pl.BlockSpec((1, Cp, _N), lambda b: (b, 0, 0)),
            pl.BlockSpec((1, _N, Cp), lambda b: (b, 0, 0)),
            pl.BlockSpec((1, _N, 128), lambda b: (b, 0, 0)),
            pl.BlockSpec((1, 8, _N), lambda b: (b, 0, 0)),
            pl.BlockSpec((o, P2), lambda b: (0, 0)),
        ],
        out_specs=[
            pl.BlockSpec((1, o, _N), lambda b: (b, 0, 0)),
            pl.BlockSpec((1, o, _N), lambda b: (b, 0, 0)),
            pl.BlockSpec((1, o, 128), lambda b: (b, 0, 0)),
            pl.BlockSpec((1, o, 128), lambda b: (b, 0, 0)),
        ],
        out_shape=[
            jax.ShapeDtypeStruct((_B, o, _N), _F32),
            jax.ShapeDtypeStruct((_B, o, _N), _F32),
            jax.ShapeDtypeStruct((_B, o, 128), _F32),
            jax.ShapeDtypeStruct((_B, o, 128), _F32),
        ],
    )(xp, xpt, xxc, xxr, wf)


def _finalize_call(zmax, zmin, m, sd, g, b, pos):
    o = zmax.shape[1]

    def bcast(v):
        return jnp.broadcast_to(v[:, None], (o, 128))

    def kern(zx_ref, zn_ref, m_ref, sd_ref, g_ref, b_ref, p_ref, out_ref):
        z = jnp.where(p_ref[:, 0:1] > 0.5, zx_ref[0], zn_ref[0])
        zn = (z - m_ref[:, 0:1]) / sd_ref[:, 0:1]
        out_ref[0] = _lrelu(zn * g_ref[:, 0:1] + b_ref[:, 0:1])

    return pl.pallas_call(
        kern,
        grid=(_B,),
        in_specs=[
            pl.BlockSpec((1, o, _N), lambda b: (b, 0, 0)),
            pl.BlockSpec((1, o, _N), lambda b: (b, 0, 0)),
            pl.BlockSpec((o, 128), lambda b: (0, 0)),
            pl.BlockSpec((o, 128), lambda b: (0, 0)),
            pl.BlockSpec((o, 128), lambda b: (0, 0)),
            pl.BlockSpec((o, 128), lambda b: (0, 0)),
            pl.BlockSpec((o, 128), lambda b: (0, 0)),
        ],
        out_specs=pl.BlockSpec((1, o, _N), lambda b: (b, 0, 0)),
        out_shape=jax.ShapeDtypeStruct((_B, o, _N), _F32),
    )(zmax, zmin, bcast(m), bcast(sd), bcast(g), bcast(b), bcast(pos))


def _global_conv_call(gft, w5p):
    Cp = gft.shape[2]
    o = w5p.shape[0]

    def kern(x_ref, w_ref, ymax_ref, ymin_ref, t1_ref, t2_ref):
        y = _d_tt(w_ref[...].astype(_BF16), x_ref[0].astype(_BF16))
        ymax_ref[0] = jnp.broadcast_to(
            jnp.max(y, axis=1, keepdims=True), (o, 128))
        ymin_ref[0] = jnp.broadcast_to(
            jnp.min(y, axis=1, keepdims=True), (o, 128))
        t1_ref[0] = jnp.broadcast_to(
            jnp.sum(y, axis=1, keepdims=True), (o, 128))
        t2_ref[0] = jnp.broadcast_to(
            jnp.sum(y * y, axis=1, keepdims=True), (o, 128))

    return pl.pallas_call(
        kern,
        grid=(_B,),
        in_specs=[
            pl.BlockSpec((1, _N, Cp), lambda b: (b, 0, 0)),
            pl.BlockSpec((o, Cp), lambda b: (0, 0)),
        ],
        out_specs=[pl.BlockSpec((1, o, 128), lambda b: (b, 0, 0))] * 4,
        out_shape=[jax.ShapeDtypeStruct((_B, o, 128), _F32)] * 4,
    )(gft, w5p)


def _head_call(pmax, pmin, m5, sd5, g5r, b5r, p5, fc1_w, fc1_b, bn1_g, bn1_b,
               fc2_w, fc2_b, bn2_g, bn2_b, fc3_wp, fc3_bp):
    def kern(pmax_ref, pmin_ref, m5_ref, sd5_ref, g5_ref, bb5_ref, p5_ref,
             w1_ref, c1_ref,
             g1_ref, d1_ref, w2_ref, c2_ref, g2_ref, d2_ref, w3_ref, c3_ref,
             logits_ref, pooled_ref):
        z = jnp.where(p5_ref[...] > 0.5, pmax_ref[...], pmin_ref[...])
        zn = (z - m5_ref[...]) / sd5_ref[...]
        pooled = _lrelu(zn * g5_ref[...] + bb5_ref[...])
        pooled_ref[...] = pooled

        def fc_bn(h, w_r, c_r, g_r, d_r):
            hh = _d_tt(h.astype(_BF16), w_r[...].astype(_BF16)) + c_r[0:1, :]
            m = jnp.mean(hh, axis=0, keepdims=True)
            v = jnp.mean((hh - m) * (hh - m), axis=0, keepdims=True)
            hn = (hh - m) / jnp.sqrt(v + _EPS)
            return _lrelu(hn * g_r[0:1, :] + d_r[0:1, :])

        h = fc_bn(pooled, w1_ref, c1_ref, g1_ref, d1_ref)
        h = fc_bn(h, w2_ref, c2_ref, g2_ref, d2_ref)
        logits = _d_tt(h.astype(_BF16),
                       w3_ref[...].astype(_BF16)) + c3_ref[0:1, :]
        logits_ref[...] = logits

    return pl.pallas_call(
        kern,
        out_shape=[
            jax.ShapeDtypeStruct((_B, 128), _F32),
            jax.ShapeDtypeStruct((_B, 1024), _F32),
        ],
    )(pmax, pmin, m5, sd5, g5r, b5r, p5, fc1_w, fc1_b, bn1_g, bn1_b,
      fc2_w, fc2_b, bn2_g, bn2_b, fc3_wp, fc3_bp)


def _pad_feats(x_in):
    C = x_in.shape[1]
    Cp = _rup(C)
    if Cp == C:
        return x_in
    return jnp.pad(x_in, ((0, 0), (0, Cp - C), (0, 0)))


def _pad_w(W):
    o, twoC = W.shape
    P2 = _rup(twoC)
    return jnp.pad(W, ((0, 0), (0, P2 - twoC)))


def _bn_stats(t1, t2, g, count):
    sum_y = jnp.sum(t1[:, :, 0], axis=0)
    sum_y2 = jnp.sum(t2[:, :, 0], axis=0)
    m = sum_y / count
    v = sum_y2 / count - m * m
    sd = jnp.sqrt(v + _EPS)
    pos = (g >= 0).astype(_F32)
    return m, sd, pos


def _edge_conv(x_in, W, g, b):
    C = x_in.shape[1]
    xp = _pad_feats(x_in)
    wf = _pad_w(W)
    xpt = jnp.transpose(xp, (0, 2, 1))
    xx = jnp.sum(xpt * xpt, axis=2, keepdims=True)
    xxc = jnp.broadcast_to(xx, (_B, _N, 128))
    xxr = jnp.broadcast_to(jnp.transpose(xx, (0, 2, 1)), (_B, 8, _N))
    zmax, zmin, t1, t2 = _edge_layer_call(xp, xpt, xxc, xxr, wf, C)
    m, sd, pos = _bn_stats(t1, t2, g, _B * _N * _K)
    return _finalize_call(zmax, zmin, m, sd, g, b, pos)


def kernel(x, W1, g1, b1, W2, g2, b2, W3, g3, b3, W4, g4, b4, W5, g5, b5,
           fc1_w, fc1_b, bn1_g, bn1_b, fc2_w, fc2_b, bn2_g, bn2_b,
           fc3_w, fc3_b):
    x3 = x[:, :3, :]
    f1 = _edge_conv(x3, W1, g1, b1)
    f2 = _edge_conv(jnp.concatenate([x3, f1], axis=1), W2, g2, b2)
    f3 = _edge_conv(jnp.concatenate([x3, f1, f2], axis=1), W3, g3, b3)
    f4 = _edge_conv(jnp.concatenate([x3, f1, f2, f3], axis=1), W4, g4, b4)
    gf = jnp.concatenate([x3, f1, f2, f3, f4], axis=1)

    gfp = _pad_feats(gf)
    Cp5 = gfp.shape[1]
    w5p = jnp.pad(W5, ((0, 0), (0, Cp5 - W5.shape[1])))
    ymax, ymin, t1, t2 = _global_conv_call(jnp.transpose(gfp, (0, 2, 1)), w5p)
    m5, sd5, p5 = _bn_stats(t1, t2, g5, _B * _N)

    pmax = ymax[:, :, 0]
    pmin = ymin[:, :, 0]

    def brow(v):
        return jnp.broadcast_to(v[None, :], (_B, 1024))

    fc3_wp = jnp.pad(fc3_w, ((0, 128 - _NUM_CLASS), (0, 0)))
    fc3_bp = jnp.broadcast_to(
        jnp.pad(fc3_b, (0, 128 - _NUM_CLASS))[None, :], (_B, 128))
    logits_p, pooled = _head_call(
        pmax, pmin, brow(m5), brow(sd5), brow(g5), brow(b5), brow(p5),
        fc1_w, jnp.broadcast_to(fc1_b[None, :], (_B, 512)),
        jnp.broadcast_to(bn1_g[None, :], (_B, 512)),
        jnp.broadcast_to(bn1_b[None, :], (_B, 512)),
        fc2_w, jnp.broadcast_to(fc2_b[None, :], (_B, 256)),
        jnp.broadcast_to(bn2_g[None, :], (_B, 256)),
        jnp.broadcast_to(bn2_b[None, :], (_B, 256)),
        fc3_wp, fc3_bp)
    return logits_p[:, :_NUM_CLASS], pooled
